# baseline (speedup 1.0000x reference)
import functools
import math

import jax
import jax.numpy as jnp
from jax import lax
from jax.experimental import pallas as pl
from jax.experimental.pallas import tpu as pltpu
from jax.experimental.pallas import tpu_sc as plsc

N = 8192
K = math.ceil(0.5 * N)
D = 256

_SC_NC = 2
_SC_NS = 16
_SC_NW = _SC_NC * _SC_NS


def _attn_mean_body(a_ref, out_ref):
    acc = a_ref[0]
    for i in range(1, 16):
        acc = acc + a_ref[i]
    v = acc[0:8]
    for g in range(1, 8):
        v = v + acc[8 * g : 8 * g + 8]
    t = v[0:4] + v[4:8]
    t = t[0:2] + t[2:4]
    t = t[0:1] + t[1:2]
    out_ref[...] = t * (1.0 / 1024.0)


def _attn_mean(A):
    LC = 1024
    return pl.pallas_call(
        _attn_mean_body,
        grid=(N // LC,),
        in_specs=[pl.BlockSpec((16, 64, LC), lambda l: (0, 0, l))],
        out_specs=pl.BlockSpec((1, LC), lambda l: (0, l)),
        out_shape=jax.ShapeDtypeStruct((1, N), jnp.float32),
    )(A)


_IB = 512
_JC = 1024


def _rank_body(cls_ref, clsT_ref, ranks_ref, w_ref):
    ib = pl.program_id(0)
    jc = pl.program_id(1)
    nj = pl.num_programs(1)
    vi = clsT_ref[...]
    vj = cls_ref[...]
    ones = jnp.ones((_JC, 1), jnp.float32)
    part = lax.dot_general(
        (vj > vi).astype(jnp.float32), ones, (((1,), (0,)), ((), ())),
        preferred_element_type=jnp.float32,
    )

    @pl.when(jc == 0)
    def _():
        ranks_ref[...] = jnp.zeros_like(ranks_ref)

    ranks_ref[...] += part

    @pl.when(2 * (jc + 1) <= ib)
    def _():
        ranks_ref[...] += lax.dot_general(
            (vj == vi).astype(jnp.float32), ones, (((1,), (0,)), ((), ())),
            preferred_element_type=jnp.float32,
        )

    @pl.when(jc == ib // 2)
    def _():
        ig = ib * _IB + lax.broadcasted_iota(jnp.int32, (_IB, _JC), 0)
        jg = jc * _JC + lax.broadcasted_iota(jnp.int32, (_IB, _JC), 1)
        tie = (vj == vi) & (jg < ig)
        ranks_ref[...] += lax.dot_general(
            tie.astype(jnp.float32), ones, (((1,), (0,)), ((), ())),
            preferred_element_type=jnp.float32,
        )

    @pl.when(jc == nj - 1)
    def _():
        w_ref[...] = jnp.where(ranks_ref[...] >= K, vi, 0.0)


def _ranks(cls_row, cls_col):
    return pl.pallas_call(
        _rank_body,
        grid=(N // _IB, N // _JC),
        in_specs=[
            pl.BlockSpec((1, _JC), lambda i, j: (0, j)),
            pl.BlockSpec((_IB, 1), lambda i, j: (i, 0)),
        ],
        out_specs=[
            pl.BlockSpec((_IB, 1), lambda i, j: (i, 0)),
            pl.BlockSpec((_IB, 1), lambda i, j: (i, 0)),
        ],
        out_shape=[
            jax.ShapeDtypeStruct((N, 1), jnp.float32),
            jax.ShapeDtypeStruct((N, 1), jnp.float32),
        ],
    )(cls_row, cls_col)


_RB = 512
_IC = 1024


def _idx_body(ranks_ref, idx_ref):
    rb = pl.program_id(0)
    ic = pl.program_id(1)
    rk = ranks_ref[...]
    rg = (rb * _RB + lax.broadcasted_iota(jnp.int32, (_RB, _IC), 0)).astype(
        jnp.float32
    )
    igcol = (
        ic * _IC + lax.broadcasted_iota(jnp.int32, (_IC, 1), 0)
    ).astype(jnp.float32)
    eq = (rk == rg).astype(jnp.float32)
    part = lax.dot_general(
        eq, igcol, (((1,), (0,)), ((), ())),
        precision=lax.Precision.HIGHEST,
        preferred_element_type=jnp.float32,
    )

    @pl.when(ic == 0)
    def _():
        idx_ref[...] = jnp.zeros_like(idx_ref)

    idx_ref[...] += part


def _topk_idx(ranks_row):
    return pl.pallas_call(
        _idx_body,
        grid=(K // _RB, N // _IC),
        in_specs=[pl.BlockSpec((1, _IC), lambda r, i: (0, i))],
        out_specs=pl.BlockSpec((_RB, 1), lambda r, i: (r, 0)),
        out_shape=jax.ShapeDtypeStruct((K, 1), jnp.float32),
    )(ranks_row)


_HB = 1024


def _inatten_body(w_ref, h_ref, out_ref):
    p = pl.program_id(0)
    part = jnp.sum(h_ref[...] * w_ref[...], axis=0, keepdims=True)

    @pl.when(p == 0)
    def _():
        out_ref[...] = jnp.zeros_like(out_ref)

    out_ref[...] += part


def _inatten(w_col, h2):
    return pl.pallas_call(
        _inatten_body,
        grid=(N // _HB,),
        in_specs=[
            pl.BlockSpec((_HB, 1), lambda i: (i, 0)),
            pl.BlockSpec((_HB, D), lambda i: (i, 0)),
        ],
        out_specs=pl.BlockSpec((1, D), lambda i: (0, 0)),
        out_shape=jax.ShapeDtypeStruct((1, D), jnp.float32),
    )(w_col, h2)


def _make_sc_gather(B, d):
    b_per_w = B // _SC_NW
    mesh = plsc.VectorSubcoreMesh(
        core_axis_name="c", subcore_axis_name="s",
        num_cores=_SC_NC, num_subcores=_SC_NS,
    )

    @functools.partial(
        pl.kernel,
        out_type=jax.ShapeDtypeStruct((B, d), jnp.float32),
        mesh=mesh,
        scratch_types=[
            pltpu.VMEM((b_per_w,), jnp.int32),
            pltpu.VMEM((b_per_w, d), jnp.float32),
            pltpu.SemaphoreType.DMA,
        ],
    )
    def gather(table_hbm, idx_hbm, out_hbm, idx_v, rows_v, sem):
        wid = lax.axis_index("s") * _SC_NC + lax.axis_index("c")
        base = wid * b_per_w
        pltpu.sync_copy(idx_hbm.at[pl.ds(base, b_per_w)], idx_v)
        pltpu.async_copy(table_hbm.at[idx_v], rows_v, sem).wait()
        pltpu.sync_copy(rows_v, out_hbm.at[pl.ds(base, b_per_w)])

    return gather


_NB = 512


def _normalize_body(x_ref, out_ref):
    x = x_ref[...]
    ss = jnp.sum(x * x, axis=1, keepdims=True)
    out_ref[...] = (x / jnp.sqrt(ss)).astype(jnp.bfloat16)


def _normalize(x):
    return pl.pallas_call(
        _normalize_body,
        grid=(K // _NB,),
        in_specs=[pl.BlockSpec((_NB, D), lambda i: (i, 0))],
        out_specs=pl.BlockSpec((_NB, D), lambda i: (i, 0)),
        out_shape=jax.ShapeDtypeStruct((K, D), jnp.bfloat16),
    )(x)


_MB = 512
_MJ = 512
_BIGI = 2**30


def _argmax_body(nv_i_ref, nv_ref, out_ref):
    ib = pl.program_id(0)
    vi = nv_i_ref[...]
    iota0 = lax.broadcasted_iota(jnp.int32, (_MB, _MJ), 0)
    iota1 = lax.broadcasted_iota(jnp.int32, (_MB, _MJ), 1)
    ig = ib * _MB + iota0

    def make_step(mask_diag):
        def step(j, carry):
            rmax, rarg = carry
            vj = nv_ref[pl.ds(j * _MJ, _MJ), :]
            g = lax.dot_general(
                vi, vj, (((1,), (1,)), ((), ())),
                preferred_element_type=jnp.float32,
            )
            jg = j * _MJ + iota1
            if mask_diag:
                g = jnp.where(ig == jg, 0.0, g)
            rowmax = jnp.max(g, axis=1, keepdims=True)
            cand = jnp.min(
                jnp.where(g == rowmax, jg, _BIGI), axis=1, keepdims=True
            )
            upd = rowmax > rmax
            return jnp.where(upd, rowmax, rmax), jnp.where(upd, cand, rarg)

        return step

    rmax0 = jnp.full((_MB, 1), -jnp.inf, jnp.float32)
    rarg0 = jnp.zeros((_MB, 1), jnp.int32)
    carry = lax.fori_loop(0, ib, make_step(False), (rmax0, rarg0))
    carry = make_step(True)(ib, carry)
    _, rarg = lax.fori_loop(ib + 1, K // _MJ, make_step(False), carry)
    out_ref[...] = rarg


def _cos_argmax(nvb):
    return pl.pallas_call(
        _argmax_body,
        grid=(K // _MB,),
        in_specs=[
            pl.BlockSpec((_MB, D), lambda i: (i, 0)),
            pl.BlockSpec((K, D), lambda i: (0, 0)),
        ],
        out_specs=pl.BlockSpec((_MB, 1), lambda i: (i, 0)),
        out_shape=jax.ShapeDtypeStruct((K, 1), jnp.int32),
    )(nvb, nvb)


_CB = 1024


def _combine_body(v_ref, a_ref, out_ref):
    out_ref[...] = 0.5 * v_ref[...] + 0.5 * a_ref[...]


def _combine(v, a):
    return pl.pallas_call(
        _combine_body,
        grid=(K // _CB,),
        in_specs=[
            pl.BlockSpec((_CB, D), lambda i: (i, 0)),
            pl.BlockSpec((_CB, D), lambda i: (i, 0)),
        ],
        out_specs=pl.BlockSpec((_CB, D), lambda i: (i, 0)),
        out_shape=jax.ShapeDtypeStruct((K, D), jnp.float32),
    )(v, a)


def kernel(h_path, A):
    h2 = h_path[0]
    cls_row = _attn_mean(A)
    cls_col = cls_row.reshape(N, 1)
    ranks_col, w_col = _ranks(cls_row, cls_col)
    idx = _topk_idx(ranks_col.reshape(1, N)).reshape(K).astype(jnp.int32)
    x_inatten = _inatten(w_col, h2)
    x_atten = _make_sc_gather(K, D)(h2, idx)
    nvb = _normalize(x_atten)
    max_list = _cos_argmax(nvb).reshape(K)
    add_vectors = _make_sc_gather(K, D)(x_atten, max_list)
    x_new = _combine(x_atten, add_vectors)
    return jnp.concatenate(
        [x_inatten[None], x_atten[None], x_new[None]], axis=1
    )


# speedup vs baseline: 1.3889x; 1.3889x over previous
import functools
import math

import jax
import jax.numpy as jnp
from jax import lax
from jax.experimental import pallas as pl
from jax.experimental.pallas import tpu as pltpu
from jax.experimental.pallas import tpu_sc as plsc

N = 8192
K = math.ceil(0.5 * N)
D = 256

_SC_NC = 2
_SC_NS = 16
_SC_NW = _SC_NC * _SC_NS


def _attn_mean_body(a_ref, out_ref):
    acc = a_ref[0]
    for i in range(1, 16):
        acc = acc + a_ref[i]
    v = acc[0:8]
    for g in range(1, 8):
        v = v + acc[8 * g : 8 * g + 8]
    t = v[0:4] + v[4:8]
    t = t[0:2] + t[2:4]
    t = t[0:1] + t[1:2]
    out_ref[...] = t * (1.0 / 1024.0)


def _attn_mean(A):
    LC = 1024
    return pl.pallas_call(
        _attn_mean_body,
        grid=(N // LC,),
        in_specs=[pl.BlockSpec((16, 64, LC), lambda l: (0, 0, l))],
        out_specs=pl.BlockSpec((1, LC), lambda l: (0, l)),
        out_shape=jax.ShapeDtypeStruct((1, N), jnp.float32),
    )(A)


_IB = 512
_JC = 1024


def _rank_body(cls_ref, clsT_ref, jrow_ref, icol_ref, ranks_ref, w_ref, acc_ref):
    ib = pl.program_id(0)
    jc = pl.program_id(1)
    nj = pl.num_programs(1)
    vi = clsT_ref[...]
    vj = cls_ref[...]

    @pl.when(jc == 0)
    def _():
        acc_ref[...] = jnp.zeros_like(acc_ref)

    @pl.when(2 * (jc + 1) <= ib)
    def _():
        acc_ref[...] += (vj >= vi).astype(jnp.int32)

    @pl.when(2 * jc >= ib + 1)
    def _():
        acc_ref[...] += (vj > vi).astype(jnp.int32)

    @pl.when(jc == ib // 2)
    def _():
        jlt = jrow_ref[...] < icol_ref[...]
        hit = (vj > vi) | ((vj == vi) & jlt)
        acc_ref[...] += hit.astype(jnp.int32)

    @pl.when(jc == nj - 1)
    def _():
        rk = jnp.sum(acc_ref[...], axis=1, keepdims=True)
        ranks_ref[...] = rk
        w_ref[...] = jnp.where(rk >= K, vi, 0.0)


def _ranks(cls_row, cls_col, arange_row, arange_col):
    return pl.pallas_call(
        _rank_body,
        grid=(N // _IB, N // _JC),
        in_specs=[
            pl.BlockSpec((1, _JC), lambda i, j: (0, j)),
            pl.BlockSpec((_IB, 1), lambda i, j: (i, 0)),
            pl.BlockSpec((1, _JC), lambda i, j: (0, j)),
            pl.BlockSpec((_IB, 1), lambda i, j: (i, 0)),
        ],
        out_specs=[
            pl.BlockSpec((_IB, 1), lambda i, j: (i, 0)),
            pl.BlockSpec((_IB, 1), lambda i, j: (i, 0)),
        ],
        out_shape=[
            jax.ShapeDtypeStruct((N, 1), jnp.int32),
            jax.ShapeDtypeStruct((N, 1), jnp.float32),
        ],
        scratch_shapes=[pltpu.VMEM((_IB, _JC), jnp.int32)],
    )(cls_row, cls_col, arange_row, arange_col)


_RB = 512
_IC = 1024


def _idx_body(ranks_ref, irow_ref, rcol_ref, idx_ref, acc_ref):
    ic = pl.program_id(1)
    nc = pl.num_programs(1)
    rk = ranks_ref[...]
    rg = rcol_ref[...]
    ig = irow_ref[...]

    @pl.when(ic == 0)
    def _():
        acc_ref[...] = jnp.zeros_like(acc_ref)

    acc_ref[...] += jnp.where(rk == rg, ig, 0)

    @pl.when(ic == nc - 1)
    def _():
        idx_ref[...] = jnp.sum(acc_ref[...], axis=1, keepdims=True)


def _topk_idx(ranks_row, arange_row, arange_col):
    return pl.pallas_call(
        _idx_body,
        grid=(K // _RB, N // _IC),
        in_specs=[
            pl.BlockSpec((1, _IC), lambda r, i: (0, i)),
            pl.BlockSpec((1, _IC), lambda r, i: (0, i)),
            pl.BlockSpec((_RB, 1), lambda r, i: (r, 0)),
        ],
        out_specs=pl.BlockSpec((_RB, 1), lambda r, i: (r, 0)),
        out_shape=jax.ShapeDtypeStruct((K, 1), jnp.int32),
        scratch_shapes=[pltpu.VMEM((_RB, _IC), jnp.int32)],
    )(ranks_row, arange_row, arange_col)


_HB = 1024


def _inatten_body(w_ref, h_ref, out_ref):
    p = pl.program_id(0)
    part = jnp.sum(h_ref[...] * w_ref[...], axis=0, keepdims=True)

    @pl.when(p == 0)
    def _():
        out_ref[...] = jnp.zeros_like(out_ref)

    out_ref[...] += part


def _inatten(w_col, h2):
    return pl.pallas_call(
        _inatten_body,
        grid=(N // _HB,),
        in_specs=[
            pl.BlockSpec((_HB, 1), lambda i: (i, 0)),
            pl.BlockSpec((_HB, D), lambda i: (i, 0)),
        ],
        out_specs=pl.BlockSpec((1, D), lambda i: (0, 0)),
        out_shape=jax.ShapeDtypeStruct((1, D), jnp.float32),
    )(w_col, h2)


def _make_sc_gather(B, d):
    b_per_w = B // _SC_NW
    mesh = plsc.VectorSubcoreMesh(
        core_axis_name="c", subcore_axis_name="s",
        num_cores=_SC_NC, num_subcores=_SC_NS,
    )

    @functools.partial(
        pl.kernel,
        out_type=jax.ShapeDtypeStruct((B, d), jnp.float32),
        mesh=mesh,
        scratch_types=[
            pltpu.VMEM((b_per_w,), jnp.int32),
            pltpu.VMEM((b_per_w, d), jnp.float32),
            pltpu.SemaphoreType.DMA,
        ],
    )
    def gather(table_hbm, idx_hbm, out_hbm, idx_v, rows_v, sem):
        wid = lax.axis_index("s") * _SC_NC + lax.axis_index("c")
        base = wid * b_per_w
        pltpu.sync_copy(idx_hbm.at[pl.ds(base, b_per_w)], idx_v)
        pltpu.async_copy(table_hbm.at[idx_v], rows_v, sem).wait()
        pltpu.sync_copy(rows_v, out_hbm.at[pl.ds(base, b_per_w)])

    return gather


_NB = 512


def _normalize_body(x_ref, out_ref):
    x = x_ref[...]
    ss = jnp.sum(x * x, axis=1, keepdims=True)
    out_ref[...] = (x / jnp.sqrt(ss)).astype(jnp.bfloat16)


def _normalize(x):
    return pl.pallas_call(
        _normalize_body,
        grid=(K // _NB,),
        in_specs=[pl.BlockSpec((_NB, D), lambda i: (i, 0))],
        out_specs=pl.BlockSpec((_NB, D), lambda i: (i, 0)),
        out_shape=jax.ShapeDtypeStruct((K, D), jnp.bfloat16),
    )(x)


_MB = 512
_MJ = 512
_BIGI = 2**30


def _argmax_body(nv_i_ref, nv_ref, out_ref):
    ib = pl.program_id(0)
    vi = nv_i_ref[...]
    iota0 = lax.broadcasted_iota(jnp.int32, (_MB, _MJ), 0)
    iota1 = lax.broadcasted_iota(jnp.int32, (_MB, _MJ), 1)
    ig = ib * _MB + iota0

    def make_step(mask_diag):
        def step(j, carry):
            rmax, rarg = carry
            vj = nv_ref[pl.ds(j * _MJ, _MJ), :]
            g = lax.dot_general(
                vi, vj, (((1,), (1,)), ((), ())),
                preferred_element_type=jnp.float32,
            )
            jg = j * _MJ + iota1
            if mask_diag:
                g = jnp.where(ig == jg, 0.0, g)
            rowmax = jnp.max(g, axis=1, keepdims=True)
            cand = jnp.min(
                jnp.where(g == rowmax, jg, _BIGI), axis=1, keepdims=True
            )
            upd = rowmax > rmax
            return jnp.where(upd, rowmax, rmax), jnp.where(upd, cand, rarg)

        return step

    rmax0 = jnp.full((_MB, 1), -jnp.inf, jnp.float32)
    rarg0 = jnp.zeros((_MB, 1), jnp.int32)
    carry = lax.fori_loop(0, ib, make_step(False), (rmax0, rarg0))
    carry = make_step(True)(ib, carry)
    _, rarg = lax.fori_loop(ib + 1, K // _MJ, make_step(False), carry)
    out_ref[...] = rarg


def _cos_argmax(nvb):
    return pl.pallas_call(
        _argmax_body,
        grid=(K // _MB,),
        in_specs=[
            pl.BlockSpec((_MB, D), lambda i: (i, 0)),
            pl.BlockSpec((K, D), lambda i: (0, 0)),
        ],
        out_specs=pl.BlockSpec((_MB, 1), lambda i: (i, 0)),
        out_shape=jax.ShapeDtypeStruct((K, 1), jnp.int32),
    )(nvb, nvb)


_CB = 1024


def _combine_body(v_ref, a_ref, out_ref):
    out_ref[...] = 0.5 * v_ref[...] + 0.5 * a_ref[...]


def _combine(v, a):
    return pl.pallas_call(
        _combine_body,
        grid=(K // _CB,),
        in_specs=[
            pl.BlockSpec((_CB, D), lambda i: (i, 0)),
            pl.BlockSpec((_CB, D), lambda i: (i, 0)),
        ],
        out_specs=pl.BlockSpec((_CB, D), lambda i: (i, 0)),
        out_shape=jax.ShapeDtypeStruct((K, D), jnp.float32),
    )(v, a)


def kernel(h_path, A):
    h2 = h_path[0]
    ar = jnp.arange(N, dtype=jnp.int32)
    cls_row = _attn_mean(A)
    cls_col = cls_row.reshape(N, 1)
    ranks_col, w_col = _ranks(
        cls_row, cls_col, ar.reshape(1, N), ar.reshape(N, 1)
    )
    idx = _topk_idx(
        ranks_col.reshape(1, N), ar.reshape(1, N), ar[:K].reshape(K, 1)
    ).reshape(K)
    x_inatten = _inatten(w_col, h2)
    x_atten = _make_sc_gather(K, D)(h2, idx)
    nvb = _normalize(x_atten)
    max_list = _cos_argmax(nvb).reshape(K)
    add_vectors = _make_sc_gather(K, D)(x_atten, max_list)
    x_new = _combine(x_atten, add_vectors)
    return jnp.concatenate(
        [x_inatten[None], x_atten[None], x_new[None]], axis=1
    )


# speedup vs baseline: 2.4591x; 1.7706x over previous
import functools
import math

import jax
import jax.numpy as jnp
from jax import lax
from jax.experimental import pallas as pl
from jax.experimental.pallas import tpu as pltpu
from jax.experimental.pallas import tpu_sc as plsc

N = 8192
K = math.ceil(0.5 * N)
D = 256

_SC_NC = 2
_SC_NS = 16
_SC_NW = _SC_NC * _SC_NS


def _attn_mean_body(a_ref, out_ref):
    acc = a_ref[0]
    for i in range(1, 16):
        acc = acc + a_ref[i]
    v = acc[0:8]
    for g in range(1, 8):
        v = v + acc[8 * g : 8 * g + 8]
    t = v[0:4] + v[4:8]
    t = t[0:2] + t[2:4]
    t = t[0:1] + t[1:2]
    out_ref[...] = t * (1.0 / 1024.0)


def _attn_mean(A):
    LC = 1024
    return pl.pallas_call(
        _attn_mean_body,
        grid=(N // LC,),
        in_specs=[pl.BlockSpec((16, 64, LC), lambda l: (0, 0, l))],
        out_specs=pl.BlockSpec((1, LC), lambda l: (0, l)),
        out_shape=jax.ShapeDtypeStruct((1, N), jnp.float32),
    )(A)


def _sort_body(cls_ref, keys_ref, idx_ref):
    rowi = lax.broadcasted_iota(jnp.int32, (64, 128), 0)
    lanei = lax.broadcasted_iota(jnp.int32, (64, 128), 1)
    f = rowi * 128 + lanei

    def exchange(keys, idxs, pk, pi, low, kbit):
        asc = (f & kbit) == 0
        mine_first = (keys > pk) | ((keys == pk) & (idxs < pi))
        sel = (low == asc) == mine_first
        return jnp.where(sel, keys, pk), jnp.where(sel, idxs, pi)

    def phase(kk, carry):
        kbit = lax.shift_left(1, kk)

        def rstage(t, c):
            keys, idxs = c
            r = lax.shift_left(1, kk - 1 - t - 7)
            low = (rowi & r) == 0
            pk = jnp.where(low, pltpu.roll(keys, -r, 0), pltpu.roll(keys, r, 0))
            pi = jnp.where(low, pltpu.roll(idxs, -r, 0), pltpu.roll(idxs, r, 0))
            return exchange(keys, idxs, pk, pi, low, kbit)

        def lstage(t, c):
            keys, idxs = c
            s = lax.shift_left(1, jnp.minimum(kk - 1, 6) - t)
            low = (lanei & s) == 0
            pk = jnp.where(low, pltpu.roll(keys, -s, 1), pltpu.roll(keys, s, 1))
            pi = jnp.where(low, pltpu.roll(idxs, -s, 1), pltpu.roll(idxs, s, 1))
            return exchange(keys, idxs, pk, pi, low, kbit)

        carry = lax.fori_loop(0, jnp.maximum(kk - 7, 0), rstage, carry)
        carry = lax.fori_loop(0, jnp.minimum(kk, 7), lstage, carry)
        return carry

    keys, idxs = lax.fori_loop(1, 14, phase, (cls_ref[...], f))
    keys_ref[...] = keys
    idx_ref[...] = idxs


def _bitonic_sort(cls64):
    return pl.pallas_call(
        _sort_body,
        out_shape=[
            jax.ShapeDtypeStruct((64, 128), jnp.float32),
            jax.ShapeDtypeStruct((64, 128), jnp.int32),
        ],
    )(cls64)


_HB = 1024


def _inatten_body(w_ref, h_ref, out_ref):
    p = pl.program_id(0)
    part = jnp.sum(h_ref[...] * w_ref[...], axis=0, keepdims=True)

    @pl.when(p == 0)
    def _():
        out_ref[...] = jnp.zeros_like(out_ref)

    out_ref[...] += part


def _inatten(w_col, h2):
    return pl.pallas_call(
        _inatten_body,
        grid=(N // _HB,),
        in_specs=[
            pl.BlockSpec((_HB, 1), lambda i: (i, 0)),
            pl.BlockSpec((_HB, D), lambda i: (i, 0)),
        ],
        out_specs=pl.BlockSpec((1, D), lambda i: (0, 0)),
        out_shape=jax.ShapeDtypeStruct((1, D), jnp.float32),
    )(w_col, h2)


def _make_sc_gather(B, d):
    b_per_w = B // _SC_NW
    mesh = plsc.VectorSubcoreMesh(
        core_axis_name="c", subcore_axis_name="s",
        num_cores=_SC_NC, num_subcores=_SC_NS,
    )

    @functools.partial(
        pl.kernel,
        out_type=jax.ShapeDtypeStruct((B, d), jnp.float32),
        mesh=mesh,
        scratch_types=[
            pltpu.VMEM((b_per_w,), jnp.int32),
            pltpu.VMEM((b_per_w, d), jnp.float32),
            pltpu.SemaphoreType.DMA,
        ],
    )
    def gather(table_hbm, idx_hbm, out_hbm, idx_v, rows_v, sem):
        wid = lax.axis_index("s") * _SC_NC + lax.axis_index("c")
        base = wid * b_per_w
        pltpu.sync_copy(idx_hbm.at[pl.ds(base, b_per_w)], idx_v)
        pltpu.async_copy(table_hbm.at[idx_v], rows_v, sem).wait()
        pltpu.sync_copy(rows_v, out_hbm.at[pl.ds(base, b_per_w)])

    return gather


_NB = 512


def _normalize_body(x_ref, w_ref, out_ref, tk_ref):
    p = pl.program_id(0)
    x = x_ref[...]
    ss = jnp.sum(x * x, axis=1, keepdims=True)
    out_ref[...] = (x / jnp.sqrt(ss)).astype(jnp.bfloat16)

    @pl.when(p == 0)
    def _():
        tk_ref[...] = jnp.zeros_like(tk_ref)

    tk_ref[...] += jnp.sum(x * w_ref[...], axis=0, keepdims=True)


def _normalize(x, w_col):
    return pl.pallas_call(
        _normalize_body,
        grid=(K // _NB,),
        in_specs=[
            pl.BlockSpec((_NB, D), lambda i: (i, 0)),
            pl.BlockSpec((_NB, 1), lambda i: (i, 0)),
        ],
        out_specs=[
            pl.BlockSpec((_NB, D), lambda i: (i, 0)),
            pl.BlockSpec((1, D), lambda i: (0, 0)),
        ],
        out_shape=[
            jax.ShapeDtypeStruct((K, D), jnp.bfloat16),
            jax.ShapeDtypeStruct((1, D), jnp.float32),
        ],
    )(x, w_col)


_MB = 512
_MJ = 512
_BIGI = 2**30


def _argmax_body(nv_i_ref, nv_ref, out_ref):
    ib = pl.program_id(0)
    vi = nv_i_ref[...]
    iota0 = lax.broadcasted_iota(jnp.int32, (_MB, _MJ), 0)
    iota1 = lax.broadcasted_iota(jnp.int32, (_MB, _MJ), 1)
    ig = ib * _MB + iota0

    def make_step(mask_diag):
        def step(j, carry):
            rmax, rarg = carry
            vj = nv_ref[pl.ds(j * _MJ, _MJ), :]
            g = lax.dot_general(
                vi, vj, (((1,), (1,)), ((), ())),
                preferred_element_type=jnp.float32,
            )
            jg = j * _MJ + iota1
            if mask_diag:
                g = jnp.where(ig == jg, 0.0, g)
            rowmax = jnp.max(g, axis=1, keepdims=True)
            cand = jnp.min(
                jnp.where(g == rowmax, jg, _BIGI), axis=1, keepdims=True
            )
            upd = rowmax > rmax
            return jnp.where(upd, rowmax, rmax), jnp.where(upd, cand, rarg)

        return step

    rmax0 = jnp.full((_MB, 1), -jnp.inf, jnp.float32)
    rarg0 = jnp.zeros((_MB, 1), jnp.int32)
    carry = lax.fori_loop(0, ib, make_step(False), (rmax0, rarg0))
    carry = make_step(True)(ib, carry)
    _, rarg = lax.fori_loop(ib + 1, K // _MJ, make_step(False), carry)
    out_ref[...] = rarg


def _cos_argmax(nvb):
    return pl.pallas_call(
        _argmax_body,
        grid=(K // _MB,),
        in_specs=[
            pl.BlockSpec((_MB, D), lambda i: (i, 0)),
            pl.BlockSpec((K, D), lambda i: (0, 0)),
        ],
        out_specs=pl.BlockSpec((_MB, 1), lambda i: (i, 0)),
        out_shape=jax.ShapeDtypeStruct((K, 1), jnp.int32),
    )(nvb, nvb)


_CB = 1024


def _combine_body(v_ref, a_ref, tot_ref, tk_ref, out_ref, xin_ref):
    out_ref[...] = 0.5 * v_ref[...] + 0.5 * a_ref[...]

    @pl.when(pl.program_id(0) == 0)
    def _():
        xin_ref[...] = tot_ref[...] - tk_ref[...]


def _combine(v, a, total, tksum):
    return pl.pallas_call(
        _combine_body,
        grid=(K // _CB,),
        in_specs=[
            pl.BlockSpec((_CB, D), lambda i: (i, 0)),
            pl.BlockSpec((_CB, D), lambda i: (i, 0)),
            pl.BlockSpec((1, D), lambda i: (0, 0)),
            pl.BlockSpec((1, D), lambda i: (0, 0)),
        ],
        out_specs=[
            pl.BlockSpec((_CB, D), lambda i: (i, 0)),
            pl.BlockSpec((1, D), lambda i: (0, 0)),
        ],
        out_shape=[
            jax.ShapeDtypeStruct((K, D), jnp.float32),
            jax.ShapeDtypeStruct((1, D), jnp.float32),
        ],
    )(v, a, total, tksum)


def kernel(h_path, A):
    h2 = h_path[0]
    cls_row = _attn_mean(A)
    cls_col = cls_row.reshape(N, 1)
    vals_s, idx_s = _bitonic_sort(cls_row.reshape(64, 128))
    idx = idx_s.reshape(N)[:K]
    w_col = vals_s.reshape(N, 1)[:K]
    total = _inatten(cls_col, h2)
    x_atten = _make_sc_gather(K, D)(h2, idx)
    nvb, tksum = _normalize(x_atten, w_col)
    max_list = _cos_argmax(nvb).reshape(K)
    add_vectors = _make_sc_gather(K, D)(x_atten, max_list)
    x_new, x_inatten = _combine(x_atten, add_vectors, total, tksum)
    return jnp.concatenate(
        [x_inatten[None], x_atten[None], x_new[None]], axis=1
    )


# speedup vs baseline: 2.7641x; 1.1240x over previous
import functools
import math

import jax
import jax.numpy as jnp
from jax import lax
from jax.experimental import pallas as pl
from jax.experimental.pallas import tpu as pltpu
from jax.experimental.pallas import tpu_sc as plsc

N = 8192
K = math.ceil(0.5 * N)
D = 256

_SC_NC = 2
_SC_NS = 16
_SC_NW = _SC_NC * _SC_NS


def _attn_mean_body(a_ref, out_ref):
    acc = a_ref[0]
    for i in range(1, 16):
        acc = acc + a_ref[i]
    v = acc[0:8]
    for g in range(1, 8):
        v = v + acc[8 * g : 8 * g + 8]
    t = v[0:4] + v[4:8]
    t = t[0:2] + t[2:4]
    t = t[0:1] + t[1:2]
    out_ref[...] = t * (1.0 / 1024.0)


def _attn_mean(A):
    LC = 1024
    return pl.pallas_call(
        _attn_mean_body,
        grid=(N // LC,),
        in_specs=[pl.BlockSpec((16, 64, LC), lambda l: (0, 0, l))],
        out_specs=pl.BlockSpec((1, LC), lambda l: (0, l)),
        out_shape=jax.ShapeDtypeStruct((1, N), jnp.float32),
    )(A)


def _sort_body(cls_ref, keys_ref, idx_ref):
    rowi = lax.broadcasted_iota(jnp.int32, (64, 128), 0)
    lanei = lax.broadcasted_iota(jnp.int32, (64, 128), 1)
    f = rowi * 128 + lanei

    def exchange(keys, idxs, pk, pi, low, kbit):
        asc = (f & kbit) == 0
        mine_first = (keys > pk) | ((keys == pk) & (idxs < pi))
        sel = (low == asc) == mine_first
        return jnp.where(sel, keys, pk), jnp.where(sel, idxs, pi)

    def phase(kk, carry):
        kbit = lax.shift_left(1, kk)

        def rstage(t, c):
            keys, idxs = c
            r = lax.shift_left(1, kk - 1 - t - 7)
            low = (rowi & r) == 0
            pk = jnp.where(low, pltpu.roll(keys, -r, 0), pltpu.roll(keys, r, 0))
            pi = jnp.where(low, pltpu.roll(idxs, -r, 0), pltpu.roll(idxs, r, 0))
            return exchange(keys, idxs, pk, pi, low, kbit)

        def lstage(t, c):
            keys, idxs = c
            s = lax.shift_left(1, jnp.minimum(kk - 1, 6) - t)
            low = (lanei & s) == 0
            pk = jnp.where(low, pltpu.roll(keys, -s, 1), pltpu.roll(keys, s, 1))
            pi = jnp.where(low, pltpu.roll(idxs, -s, 1), pltpu.roll(idxs, s, 1))
            return exchange(keys, idxs, pk, pi, low, kbit)

        carry = lax.fori_loop(0, jnp.maximum(kk - 7, 0), rstage, carry)
        carry = lax.fori_loop(0, jnp.minimum(kk, 7), lstage, carry)
        return carry

    keys, idxs = lax.fori_loop(1, 14, phase, (cls_ref[...], f))
    keys_ref[...] = keys
    idx_ref[...] = idxs


def _bitonic_sort(cls64):
    return pl.pallas_call(
        _sort_body,
        out_shape=[
            jax.ShapeDtypeStruct((64, 128), jnp.float32),
            jax.ShapeDtypeStruct((64, 128), jnp.int32),
        ],
    )(cls64)


_HB = 1024


def _inatten_body(w_ref, h_ref, out_ref):
    p = pl.program_id(0)
    part = jnp.sum(h_ref[...] * w_ref[...], axis=0, keepdims=True)

    @pl.when(p == 0)
    def _():
        out_ref[...] = jnp.zeros_like(out_ref)

    out_ref[...] += part


def _inatten(w_col, h2):
    return pl.pallas_call(
        _inatten_body,
        grid=(N // _HB,),
        in_specs=[
            pl.BlockSpec((_HB, 1), lambda i: (i, 0)),
            pl.BlockSpec((_HB, D), lambda i: (i, 0)),
        ],
        out_specs=pl.BlockSpec((1, D), lambda i: (0, 0)),
        out_shape=jax.ShapeDtypeStruct((1, D), jnp.float32),
    )(w_col, h2)


def _make_sc_gather(B, d):
    b_per_w = B // _SC_NW
    mesh = plsc.VectorSubcoreMesh(
        core_axis_name="c", subcore_axis_name="s",
        num_cores=_SC_NC, num_subcores=_SC_NS,
    )

    @functools.partial(
        pl.kernel,
        out_type=jax.ShapeDtypeStruct((B, d), jnp.float32),
        mesh=mesh,
        scratch_types=[
            pltpu.VMEM((b_per_w,), jnp.int32),
            pltpu.VMEM((b_per_w, d), jnp.float32),
            pltpu.SemaphoreType.DMA,
        ],
    )
    def gather(table_hbm, idx_hbm, out_hbm, idx_v, rows_v, sem):
        wid = lax.axis_index("s") * _SC_NC + lax.axis_index("c")
        base = wid * b_per_w
        pltpu.sync_copy(idx_hbm.at[pl.ds(base, b_per_w)], idx_v)
        pltpu.async_copy(table_hbm.at[idx_v], rows_v, sem).wait()
        pltpu.sync_copy(rows_v, out_hbm.at[pl.ds(base, b_per_w)])

    return gather


_NB = 512


def _normalize_body(x_ref, w_ref, out_ref, tk_ref):
    p = pl.program_id(0)
    x = x_ref[...]
    ss = jnp.sum(x * x, axis=1, keepdims=True)
    out_ref[...] = (x / jnp.sqrt(ss)).astype(jnp.bfloat16)

    @pl.when(p == 0)
    def _():
        tk_ref[...] = jnp.zeros_like(tk_ref)

    tk_ref[...] += jnp.sum(x * w_ref[...], axis=0, keepdims=True)


def _normalize(x, w_col):
    return pl.pallas_call(
        _normalize_body,
        grid=(K // _NB,),
        in_specs=[
            pl.BlockSpec((_NB, D), lambda i: (i, 0)),
            pl.BlockSpec((_NB, 1), lambda i: (i, 0)),
        ],
        out_specs=[
            pl.BlockSpec((_NB, D), lambda i: (i, 0)),
            pl.BlockSpec((1, D), lambda i: (0, 0)),
        ],
        out_shape=[
            jax.ShapeDtypeStruct((K, D), jnp.bfloat16),
            jax.ShapeDtypeStruct((1, D), jnp.float32),
        ],
    )(x, w_col)


_MB = 512
_MJ = 512
_BIGI = 2**30


def _argmax_body(nv_i_ref, nv_ref, out_ref):
    ib = pl.program_id(0)
    vi = nv_i_ref[...]
    iota0 = lax.broadcasted_iota(jnp.int32, (_MB, _MJ), 0).astype(jnp.float32)
    iota1 = lax.broadcasted_iota(jnp.int32, (_MB, _MJ), 1).astype(jnp.float32)
    ig = ib * _MB + iota0

    def make_step(mask_diag):
        def step(j, carry):
            rmax, rarg = carry
            vj = nv_ref[pl.ds(j * _MJ, _MJ), :]
            g = lax.dot_general(
                vi, vj, (((1,), (1,)), ((), ())),
                preferred_element_type=jnp.float32,
            )
            jg = jnp.float32(j * _MJ) + iota1
            if mask_diag:
                g = jnp.where(ig == jg, 0.0, g)
            rowmax = jnp.max(g, axis=1, keepdims=True)
            cand = jnp.min(
                jnp.where(g == rowmax, jg, jnp.inf), axis=1, keepdims=True
            )
            upd = rowmax > rmax
            return jnp.where(upd, rowmax, rmax), jnp.where(upd, cand, rarg)

        return step

    rmax0 = jnp.full((_MB, 1), -jnp.inf, jnp.float32)
    rarg0 = jnp.zeros((_MB, 1), jnp.float32)
    carry = lax.fori_loop(0, ib, make_step(False), (rmax0, rarg0))
    carry = make_step(True)(ib, carry)
    _, rarg = lax.fori_loop(ib + 1, K // _MJ, make_step(False), carry)
    out_ref[...] = rarg.astype(jnp.int32)


def _cos_argmax(nvb):
    return pl.pallas_call(
        _argmax_body,
        grid=(K // _MB,),
        in_specs=[
            pl.BlockSpec((_MB, D), lambda i: (i, 0)),
            pl.BlockSpec((K, D), lambda i: (0, 0)),
        ],
        out_specs=pl.BlockSpec((_MB, 1), lambda i: (i, 0)),
        out_shape=jax.ShapeDtypeStruct((K, 1), jnp.int32),
    )(nvb, nvb)


_AB = 512
_NAB = K // _AB


def _assemble_body(
    xa_prev_ref, xa_cur_ref, nxa_prev_ref, nav_prev_ref, nxa_cur_ref,
    nav_cur_ref, tot_ref, tk_ref, out_ref
):
    b = pl.program_id(0)

    @pl.when(b == 0)
    def _():
        xin = tot_ref[...] - tk_ref[...]
        out_ref[...] = jnp.concatenate(
            [xin, xa_cur_ref[0 : _AB - 1]], axis=0
        )

    @pl.when((b >= 1) & (b <= _NAB - 1))
    def _():
        out_ref[...] = jnp.concatenate(
            [xa_prev_ref[_AB - 1 : _AB], xa_cur_ref[0 : _AB - 1]], axis=0
        )

    @pl.when(b == _NAB)
    def _():
        tail = 0.5 * nxa_cur_ref[0 : _AB - 1] + 0.5 * nav_cur_ref[0 : _AB - 1]
        out_ref[...] = jnp.concatenate(
            [xa_prev_ref[_AB - 1 : _AB], tail], axis=0
        )

    @pl.when(b >= _NAB + 1)
    def _():
        head = (
            0.5 * nxa_prev_ref[_AB - 1 : _AB]
            + 0.5 * nav_prev_ref[_AB - 1 : _AB]
        )
        tail = 0.5 * nxa_cur_ref[0 : _AB - 1] + 0.5 * nav_cur_ref[0 : _AB - 1]
        out_ref[...] = jnp.concatenate([head, tail], axis=0)


def _assemble(x_atten, add_vectors, total, tksum):
    def clip(i, lo, hi):
        return jnp.minimum(jnp.maximum(i, lo), hi)

    return pl.pallas_call(
        _assemble_body,
        grid=(2 * K // _AB + 1,),
        in_specs=[
            pl.BlockSpec((_AB, D), lambda b: (clip(b - 1, 0, _NAB - 1), 0)),
            pl.BlockSpec((_AB, D), lambda b: (clip(b, 0, _NAB - 1), 0)),
            pl.BlockSpec((_AB, D), lambda b: (clip(b - _NAB - 1, 0, _NAB - 1), 0)),
            pl.BlockSpec((_AB, D), lambda b: (clip(b - _NAB - 1, 0, _NAB - 1), 0)),
            pl.BlockSpec((_AB, D), lambda b: (clip(b - _NAB, 0, _NAB - 1), 0)),
            pl.BlockSpec((_AB, D), lambda b: (clip(b - _NAB, 0, _NAB - 1), 0)),
            pl.BlockSpec((1, D), lambda b: (0, 0)),
            pl.BlockSpec((1, D), lambda b: (0, 0)),
        ],
        out_specs=pl.BlockSpec((_AB, D), lambda b: (b, 0)),
        out_shape=jax.ShapeDtypeStruct((1 + 2 * K, D), jnp.float32),
    )(x_atten, x_atten, x_atten, add_vectors, x_atten, add_vectors,
      total, tksum)


def kernel(h_path, A):
    h2 = h_path[0]
    cls_row = _attn_mean(A)
    cls_col = cls_row.reshape(N, 1)
    vals_s, idx_s = _bitonic_sort(cls_row.reshape(64, 128))
    idx = idx_s.reshape(N)[:K]
    w_col = vals_s.reshape(N, 1)[:K]
    total = _inatten(cls_col, h2)
    x_atten = _make_sc_gather(K, D)(h2, idx)
    nvb, tksum = _normalize(x_atten, w_col)
    max_list = _cos_argmax(nvb).reshape(K)
    add_vectors = _make_sc_gather(K, D)(x_atten, max_list)
    return _assemble(x_atten, add_vectors, total, tksum)[None]
